# baseline (speedup 1.0000x reference)
import functools

import jax
import jax.numpy as jnp
from jax import lax
from jax.experimental import pallas as pl
from jax.experimental.pallas import tpu as pltpu
from jax.experimental.pallas import tpu_sc as plsc

_EPS = 1e-12
_TH = 256
_HB = 8
_CH = 8192


def _count_body(x_ref, bot_ref, cnt_ref, carry_ref):
    h = pl.program_id(1)
    nh = pl.num_programs(1)
    x = x_ref[0]
    _, th, w = x.shape

    pred = jnp.argmax(x, axis=0).astype(jnp.int32)
    top_row = jnp.where(h == 0, -1, carry_ref[0:1, :])
    bot_row = jnp.argmax(bot_ref[0, :, 0:1, :], axis=0).astype(jnp.int32)
    bot_row = jnp.where(h == nh - 1, -1, bot_row)
    carry_ref[0:1, :] = pred[th - 1:th, :]

    padded = jnp.concatenate([top_row, pred, bot_row], axis=0)
    col = lax.broadcasted_iota(jnp.int32, (th, w), 1)

    count = jnp.zeros((th, w), dtype=jnp.int32)
    for di in (-1, 0, 1):
        rows = padded[1 + di:1 + di + th, :]
        for dj in (-1, 0, 1):
            if di == 0 and dj == 0:
                continue
            if dj == 0:
                nb = rows
            else:
                nb = jnp.roll(rows, -dj, axis=1)
                edge = w - 1 if dj == 1 else 0
                nb = jnp.where(col == edge, -1, nb)
            count = count + (nb == pred).astype(jnp.int32)
    cnt_ref[0] = count


def _scale_body(x_ref, t_ref, o_ref):
    o_ref[0] = x_ref[0] / t_ref[0][None, :, :]


def _sc_gather_body(counts_hbm, table_hbm, out_hbm, idx_v, out_v, table_v):
    info = plsc.get_sparse_core_info()
    nc = info.num_cores
    wid = lax.axis_index("s") * nc + lax.axis_index("c")
    n = counts_hbm.shape[0]
    per_w = n // (nc * info.num_subcores)
    base = wid * per_w
    pltpu.sync_copy(table_hbm, table_v)
    tbl = jnp.maximum(table_v[...], 0.0) + _EPS

    def chunk(c, _):
        off = base + c * _CH
        pltpu.sync_copy(counts_hbm.at[pl.ds(off, _CH)], idx_v)

        dnums = lax.GatherDimensionNumbers(
            offset_dims=(), collapsed_slice_dims=(0,), start_index_map=(0,))

        def body(j, _):
            idx = idx_v[pl.ds(j * 16, 16)]
            out_v[pl.ds(j * 16, 16)] = lax.gather(
                tbl, idx[:, None], dnums, (1,),
                mode=lax.GatherScatterMode.PROMISE_IN_BOUNDS)
            return _

        lax.fori_loop(0, _CH // 16, body, None)
        pltpu.sync_copy(out_v, out_hbm.at[pl.ds(off, _CH)])
        return _

    lax.fori_loop(0, per_w // _CH, chunk, None)


def _sc_gather(counts_flat, table16):
    n = counts_flat.shape[0]
    mesh = plsc.VectorSubcoreMesh(core_axis_name="c", subcore_axis_name="s")
    return pl.kernel(
        _sc_gather_body,
        mesh=mesh,
        out_type=jax.ShapeDtypeStruct((n,), jnp.float32),
        scratch_types=[
            pltpu.VMEM((_CH,), jnp.int32),
            pltpu.VMEM((_CH,), jnp.float32),
            pltpu.VMEM((16,), jnp.float32),
        ],
    )(counts_flat, table16)


def kernel(logits, neighborhood_temps):
    B, C, H, W = logits.shape
    th = min(_TH, H)
    nh = H // th
    nhb = th // _HB

    counts = pl.pallas_call(
        _count_body,
        grid=(B, nh),
        in_specs=[
            pl.BlockSpec((1, C, th, W), lambda b, h: (b, 0, h, 0)),
            pl.BlockSpec((1, C, _HB, W),
                         lambda b, h: (b, 0, jnp.minimum((h + 1) * nhb, H // _HB - 1), 0)),
        ],
        out_specs=pl.BlockSpec((1, th, W), lambda b, h: (b, h, 0)),
        out_shape=jax.ShapeDtypeStruct((B, H, W), jnp.int32),
        scratch_shapes=[pltpu.VMEM((8, W), jnp.int32)],
    )(logits, logits)

    table16 = jnp.pad(neighborhood_temps, (0, 16 - neighborhood_temps.shape[0]))
    temps = _sc_gather(counts.reshape(B * H * W), table16).reshape(B, H, W)

    return pl.pallas_call(
        _scale_body,
        grid=(B, nh),
        in_specs=[
            pl.BlockSpec((1, C, th, W), lambda b, h: (b, 0, h, 0)),
            pl.BlockSpec((1, th, W), lambda b, h: (b, h, 0)),
        ],
        out_specs=pl.BlockSpec((1, C, th, W), lambda b, h: (b, 0, h, 0)),
        out_shape=jax.ShapeDtypeStruct(logits.shape, logits.dtype),
    )(logits, temps)


# speedup vs baseline: 2.1191x; 2.1191x over previous
import jax
import jax.numpy as jnp
from jax import lax
from jax.experimental import pallas as pl
from jax.experimental.pallas import tpu as pltpu

_EPS = 1e-12
_TH = 256
_HB = 8


def _body(temps_ref, x_ref, bot_ref, o_ref, carry_ref):
    h = pl.program_id(1)
    nh = pl.num_programs(1)
    x = x_ref[0]
    _, th, w = x.shape

    pred = jnp.argmax(x, axis=0).astype(jnp.int32)
    top_row = jnp.where(h == 0, -1, carry_ref[0:1, :])
    bot_row = jnp.argmax(bot_ref[0, :, 0:1, :], axis=0).astype(jnp.int32)
    bot_row = jnp.where(h == nh - 1, -1, bot_row)
    carry_ref[0:1, :] = pred[th - 1:th, :]

    padded = jnp.concatenate([top_row, pred, bot_row], axis=0)
    col = lax.broadcasted_iota(jnp.int32, (th, w), 1)

    count = jnp.zeros((th, w), dtype=jnp.int32)
    for di in (-1, 0, 1):
        rows = padded[1 + di:1 + di + th, :]
        for dj in (-1, 0, 1):
            if di == 0 and dj == 0:
                continue
            if dj == 0:
                nb = rows
            else:
                nb = jnp.roll(rows, -dj, axis=1)
                edge = w - 1 if dj == 1 else 0
                nb = jnp.where(col == edge, -1, nb)
            count = count + (nb == pred).astype(jnp.int32)

    temp = jnp.full((th, w), temps_ref[0], dtype=jnp.float32)
    for k in range(1, 9):
        temp = jnp.where(count == k, temps_ref[k], temp)

    t = jnp.maximum(temp, 0.0) + _EPS
    o_ref[0] = x / t[None, :, :]


def kernel(logits, neighborhood_temps):
    B, C, H, W = logits.shape
    th = min(_TH, H)
    nh = H // th
    nhb = th // _HB
    return pl.pallas_call(
        _body,
        grid=(B, nh),
        in_specs=[
            pl.BlockSpec(memory_space=pltpu.SMEM),
            pl.BlockSpec((1, C, th, W), lambda b, h: (b, 0, h, 0)),
            pl.BlockSpec((1, C, _HB, W),
                         lambda b, h: (b, 0, jnp.minimum((h + 1) * nhb, H // _HB - 1), 0)),
        ],
        out_specs=pl.BlockSpec((1, C, th, W), lambda b, h: (b, 0, h, 0)),
        out_shape=jax.ShapeDtypeStruct(logits.shape, logits.dtype),
        scratch_shapes=[pltpu.VMEM((8, W), jnp.int32)],
    )(neighborhood_temps, logits, logits)


# speedup vs baseline: 2.1206x; 1.0007x over previous
import jax
import jax.numpy as jnp
from jax import lax
from jax.experimental import pallas as pl
from jax.experimental.pallas import tpu as pltpu

_EPS = 1e-12
_TH = 256
_HB = 8


def _body(temps_ref, x_ref, bot_ref, o_ref, carry_ref):
    h = pl.program_id(1)
    nh = pl.num_programs(1)
    x = x_ref[0]
    _, th, w = x.shape

    pred = jnp.argmax(x, axis=0).astype(jnp.int32)
    top_row = jnp.where(h == 0, -1, carry_ref[0:1, :])
    bot_row = jnp.argmax(bot_ref[0, :, 0:1, :], axis=0).astype(jnp.int32)
    bot_row = jnp.where(h == nh - 1, -1, bot_row)
    carry_ref[0:1, :] = pred[th - 1:th, :]

    padded = jnp.concatenate([top_row, pred, bot_row], axis=0)
    col = lax.broadcasted_iota(jnp.int32, (th, w), 1)

    count = jnp.zeros((th, w), dtype=jnp.int32)
    for di in (-1, 0, 1):
        rows = padded[1 + di:1 + di + th, :]
        for dj in (-1, 0, 1):
            if di == 0 and dj == 0:
                continue
            if dj == 0:
                nb = rows
            else:
                nb = jnp.roll(rows, -dj, axis=1)
                edge = w - 1 if dj == 1 else 0
                nb = jnp.where(col == edge, -1, nb)
            count = count + (nb == pred).astype(jnp.int32)

    temp = jnp.full((th, w), temps_ref[0], dtype=jnp.float32)
    for k in range(1, 9):
        temp = jnp.where(count == k, temps_ref[k], temp)

    t = jnp.maximum(temp, 0.0) + _EPS
    inv = 1.0 / t
    o_ref[0] = x * inv[None, :, :]


def kernel(logits, neighborhood_temps):
    B, C, H, W = logits.shape
    th = min(_TH, H)
    nh = H // th
    nhb = th // _HB
    return pl.pallas_call(
        _body,
        grid=(B, nh),
        in_specs=[
            pl.BlockSpec(memory_space=pltpu.SMEM),
            pl.BlockSpec((1, C, th, W), lambda b, h: (b, 0, h, 0)),
            pl.BlockSpec((1, C, _HB, W),
                         lambda b, h: (b, 0, jnp.minimum((h + 1) * nhb, H // _HB - 1), 0)),
        ],
        out_specs=pl.BlockSpec((1, C, th, W), lambda b, h: (b, 0, h, 0)),
        out_shape=jax.ShapeDtypeStruct(logits.shape, logits.dtype),
        scratch_shapes=[pltpu.VMEM((8, W), jnp.int32)],
    )(neighborhood_temps, logits, logits)
